# Initial kernel scaffold; baseline (speedup 1.0000x reference)
import jax
import jax.numpy as jnp
from jax.experimental import pallas as pl


def kernel(x, edge_index, batch, W1, b1, W2, b2, W3, b3, W4, b4, p1_Wr, p1_br, p1_Wroot, p2_Wr, p2_br, p2_Wroot, p3_Wr, p3_br, p3_Wroot):
    raise NotImplementedError("write your pallas kernel here")


# speedup vs baseline: 1.0311x; 1.0311x over previous
import functools
import jax
import jax.numpy as jnp
from jax.experimental import pallas as pl
from jax.experimental.pallas import tpu as pltpu

_N = 10000
_E = 150000


def _mm_body(h_ref, w_ref, o_ref):
    o_ref[...] = jnp.dot(h_ref[...], w_ref[...],
                         preferred_element_type=jnp.float32)


def _mm(h, w, block_rows=1000):
    n, k = h.shape
    ko, m = w.shape
    assert k == ko and n % block_rows == 0
    return pl.pallas_call(
        _mm_body,
        grid=(n // block_rows,),
        in_specs=[
            pl.BlockSpec((block_rows, k), lambda i: (i, 0)),
            pl.BlockSpec((k, m), lambda i: (0, 0)),
        ],
        out_specs=pl.BlockSpec((block_rows, m), lambda i: (i, 0)),
        out_shape=jax.ShapeDtypeStruct((n, m), jnp.float32),
    )(h, w)


def _gcn_conv(h, src, dst, ew, W, b):
    n = h.shape[0]
    xw = _mm(h, W)
    deg = jnp.zeros((n,), jnp.float32).at[dst].add(ew)
    dinv = jnp.where(deg > 0, 1.0 / jnp.sqrt(deg), 0.0)
    norm = dinv[src] * dinv[dst] * ew
    perm = jnp.argsort(dst, stable=True)
    agg = jnp.zeros((n, W.shape[1]), jnp.float32).at[dst[perm]].add(
        xw[src[perm]] * norm[perm][:, None])
    return agg + b


def _pool(h, src, dst, ew, valid, k, Wr, br, Wroot):
    n = h.shape[0]
    agg_feat = jnp.zeros((n, h.shape[1]), jnp.float32).at[dst].add(
        h[src] * ew[:, None])
    wr_pad = jnp.pad(Wr, ((0, 0), (0, 127)))
    wroot_pad = jnp.pad(Wroot, ((0, 0), (0, 127)))
    score = (_mm(agg_feat, wr_pad)[:, 0] + br[0]
             + _mm(h, wroot_pad)[:, 0])
    mscore = jnp.where(valid, score, -jnp.inf)
    kth = jax.lax.top_k(mscore, k)[0][k - 1]
    gt = mscore > kth
    n_gt = jnp.sum(gt.astype(jnp.int32))
    tie = mscore == kth
    tie_rank = jnp.cumsum(tie.astype(jnp.int32)) - 1
    sel = gt | (tie & (tie_rank < (k - n_gt)))
    h_new = jnp.where(sel[:, None], h * jnp.tanh(score)[:, None], 0.0)
    ew_new = ew * sel[src].astype(jnp.float32) * sel[dst].astype(jnp.float32)
    return h_new, ew_new, sel


def kernel(x, edge_index, batch, W1, b1, W2, b2, W3, b3, W4, b4,
           p1_Wr, p1_br, p1_Wroot, p2_Wr, p2_br, p2_Wroot,
           p3_Wr, p3_br, p3_Wroot):
    src = edge_index[0]
    dst = edge_index[1]
    ew = jnp.ones((_E,), jnp.float32)
    valid = jnp.ones((_N,), bool)

    h = _gcn_conv(x, src, dst, ew, W1, b1)
    h = jax.nn.leaky_relu(h, 0.01)
    h, ew, valid = _pool(h, src, dst, ew, valid, 6000, p1_Wr, p1_br, p1_Wroot)

    h = _gcn_conv(h, src, dst, ew, W2, b2)
    h = jax.nn.leaky_relu(h, 0.01)
    h, ew, valid = _pool(h, src, dst, ew, valid, 3600, p2_Wr, p2_br, p2_Wroot)

    h = _gcn_conv(h, src, dst, ew, W3, b3)
    h = jax.nn.leaky_relu(h, 0.01)
    h, ew, valid = _pool(h, src, dst, ew, valid, 1800, p3_Wr, p3_br, p3_Wroot)

    x2 = _gcn_conv(h, src, dst, ew, W4, b4)

    vm = valid[:, None]
    gmax = jnp.max(jnp.where(vm, x2, -jnp.inf), axis=0, keepdims=True)
    gmean = jnp.sum(jnp.where(vm, x2, 0.0), axis=0, keepdims=True) / 1800.0
    return jnp.concatenate([gmax, gmean], axis=1)


# speedup vs baseline: 1.6689x; 1.6186x over previous
import functools

import jax
import jax.numpy as jnp
from jax import lax
from jax.experimental import pallas as pl
from jax.experimental.pallas import tpu as pltpu
from jax.experimental.pallas import tpu_sc as plsc

_N = 10000
_E = 150000
_NP = 10240
_ROWS = 320
_D = 256
_C = 32
_MB = 5120
_EPAD = _E + 2 * _MB


def _mm_body(h_ref, w_ref, o_ref):
    o_ref[...] = jnp.dot(h_ref[...], w_ref[...],
                         preferred_element_type=jnp.float32)


def _mm(h, w, block_rows=1000):
    n, k = h.shape
    ko, m = w.shape
    assert k == ko and n % block_rows == 0
    return pl.pallas_call(
        _mm_body,
        grid=(n // block_rows,),
        in_specs=[
            pl.BlockSpec((block_rows, k), lambda i: (i, 0)),
            pl.BlockSpec((k, m), lambda i: (0, 0)),
        ],
        out_specs=pl.BlockSpec((block_rows, m), lambda i: (i, 0)),
        out_shape=jax.ShapeDtypeStruct((n, m), jnp.float32),
    )(h, w)


def _sc_agg_body(y_h, src_h, dstn_h, w_h, dinv_h, rp_h, out_h,
                 idx_m, dstn_m, w_m, rows_a, rows_b, dinv_v, rp_v, out_v,
                 sem_a, sem_b):
    wid = lax.axis_index("s") * 2 + lax.axis_index("c")
    n0 = wid * _ROWS
    pltpu.sync_copy(dinv_h, dinv_v)
    pltpu.sync_copy(rp_h.at[pl.ds(n0, _ROWS + 16)], rp_v)
    e0 = rp_v[pl.ds(0, 16)][0]
    e1 = rp_v[pl.ds(_ROWS, 16)][0]

    zero = jnp.zeros((16,), jnp.float32)

    def zrow(r, carry):
        for j in range(_D // 16):
            out_v[r, pl.ds(j * 16, 16)] = zero
        return carry

    lax.fori_loop(0, _ROWS + 1, zrow, 0)

    nblk = lax.div(e1 - e0 + (_MB - 1), _MB)

    def consume(cb, lo, hi, rows_v, m0):
        for g in range(_C // 16):
            gb = cb + 16 * g
            idx16 = idx_m[pl.ds(gb - m0, 16)]
            dn16 = dstn_m[pl.ds(gb - m0, 16)]
            w16 = w_m[pl.ds(gb - m0, 16)]
            eabs = lax.broadcasted_iota(jnp.int32, (16,), 0) + gb
            act = (eabs >= lo) & (eabs < hi)
            dl16 = jnp.where(act, dn16 - n0, _ROWS)
            for j in range(16):
                dl = dl16[j]
                ds = dinv_v[pl.ds(idx16[j], 16)][0]
                dd = dinv_v[pl.ds(dn16[j], 16)][0]
                weff = (ds * dd) * w16[j]
                wb = jnp.full((16,), weff, jnp.float32)
                for q in range(_D // 16):
                    sl = pl.ds(q * 16, 16)
                    out_v[dl, sl] = out_v[dl, sl] + rows_v[16 * g + j, sl] * wb

    def blk_body(b, carry):
        mlo = e0 + b * _MB
        mhi = jnp.minimum(e1, mlo + _MB)
        m0 = lax.div(mlo, 2 * _C) * (2 * _C)
        pltpu.sync_copy(src_h.at[pl.ds(m0, _MB + 128)], idx_m)
        pltpu.sync_copy(dstn_h.at[pl.ds(m0, _MB + 128)], dstn_m)
        pltpu.sync_copy(w_h.at[pl.ds(m0, _MB + 128)], w_m)

        p0 = lax.div(mlo, 2 * _C)
        p1 = lax.div(mhi + (2 * _C - 1), 2 * _C)

        def pair(p, carry):
            cba = p * (2 * _C)
            cbb = cba + _C
            da = pltpu.async_copy(
                y_h.at[idx_m.at[pl.ds(cba - m0, _C)]], rows_a, sem_a)
            db = pltpu.async_copy(
                y_h.at[idx_m.at[pl.ds(cbb - m0, _C)]], rows_b, sem_b)
            da.wait()
            consume(cba, jnp.maximum(mlo, cba), jnp.minimum(mhi, cba + _C),
                    rows_a, m0)
            db.wait()
            consume(cbb, jnp.maximum(mlo, cbb), jnp.minimum(mhi, cbb + _C),
                    rows_b, m0)
            return carry

        lax.fori_loop(p0, p1, pair, 0)
        return carry

    lax.fori_loop(0, nblk, blk_body, 0)
    pltpu.sync_copy(out_v.at[pl.ds(0, _ROWS)], out_h.at[pl.ds(n0, _ROWS)])


_sc_agg_call = functools.partial(
    pl.kernel,
    out_type=jax.ShapeDtypeStruct((_NP, _D), jnp.float32),
    mesh=plsc.VectorSubcoreMesh(core_axis_name="c", subcore_axis_name="s"),
    scratch_types=[
        pltpu.VMEM((_MB + 128,), jnp.int32),
        pltpu.VMEM((_MB + 128,), jnp.int32),
        pltpu.VMEM((_MB + 128,), jnp.float32),
        pltpu.VMEM((_C, _D), jnp.float32),
        pltpu.VMEM((_C, _D), jnp.float32),
        pltpu.VMEM((_NP + 16,), jnp.float32),
        pltpu.VMEM((_ROWS + 16,), jnp.int32),
        pltpu.VMEM((_ROWS + 1, _D), jnp.float32),
        pltpu.SemaphoreType.DMA,
        pltpu.SemaphoreType.DMA,
    ],
)(_sc_agg_body)


def _sc_agg(y, csr_src, csr_dstn, csr_w, dinv_np, rowptr):
    w_pad = jnp.pad(csr_w, (0, _EPAD - _E))
    out = _sc_agg_call(y, csr_src, csr_dstn, w_pad, dinv_np, rowptr)
    return out[:_N]


def _agg_wide(h, csr_src, csr_dstn, csr_w, dinv_np, rowptr):
    d = h.shape[1]
    if d == 256:
        return _sc_agg(h, csr_src, csr_dstn, csr_w, dinv_np, rowptr)
    parts = [
        _sc_agg(h[:, i * 256:(i + 1) * 256], csr_src, csr_dstn, csr_w,
                dinv_np, rowptr)
        for i in range(d // 256)
    ]
    return jnp.concatenate(parts, axis=1)


def _gcn_conv(h, G, ew_csr, deg, W, b):
    dinv = jnp.where(deg > 0, 1.0 / jnp.sqrt(deg), 0.0)
    dinv_np = jnp.pad(dinv, (0, _NP + 16 - _N))
    xw = _mm(h, W)
    agg = _agg_wide(xw, G["src"], G["dstn"], ew_csr, dinv_np, G["rowptr"])
    return agg + b


def _pool(h, G, ew_csr, valid, k, Wr, br, Wroot):
    ones_np = jnp.ones((_NP + 16,), jnp.float32)
    agg_feat = _agg_wide(h, G["src"], G["dstn"], ew_csr, ones_np,
                         G["rowptr"])
    wr_pad = jnp.pad(Wr, ((0, 0), (0, 127)))
    wroot_pad = jnp.pad(Wroot, ((0, 0), (0, 127)))
    score = (_mm(agg_feat, wr_pad)[:, 0] + br[0]
             + _mm(h, wroot_pad)[:, 0])
    mscore = jnp.where(valid, score, -jnp.inf)
    kth = lax.top_k(mscore, k)[0][k - 1]
    gt = mscore > kth
    n_gt = jnp.sum(gt.astype(jnp.int32))
    tie = mscore == kth
    tie_rank = jnp.cumsum(tie.astype(jnp.int32)) - 1
    sel = gt | (tie & (tie_rank < (k - n_gt)))
    h_new = jnp.where(sel[:, None], h * jnp.tanh(score)[:, None], 0.0)
    sel_f = sel.astype(jnp.float32)
    ew_new = ew_csr * sel_f[G["src"][:_E]] * sel_f[G["dstn_real"]]
    cs = jnp.concatenate([jnp.zeros((1,), jnp.float32), jnp.cumsum(ew_new)])
    deg = cs[G["rowptr"][1:_N + 1]] - cs[G["rowptr"][:_N]]
    return h_new, ew_new, sel, deg


def kernel(x, edge_index, batch, W1, b1, W2, b2, W3, b3, W4, b4,
           p1_Wr, p1_br, p1_Wroot, p2_Wr, p2_br, p2_Wroot,
           p3_Wr, p3_br, p3_Wroot):
    src = edge_index[0]
    dst = edge_index[1]

    perm = jnp.argsort(dst, stable=True)
    csr_src_r = src[perm]
    csr_dstn_r = dst[perm]
    pad_i = jnp.zeros((_EPAD - _E,), jnp.int32)
    G = {
        "src": jnp.concatenate([csr_src_r, pad_i]),
        "dstn": jnp.concatenate([csr_dstn_r, pad_i]),
        "dstn_real": csr_dstn_r,
        "rowptr": jnp.searchsorted(
            csr_dstn_r, jnp.arange(_NP + 16, dtype=jnp.int32)
        ).astype(jnp.int32),
    }
    ew = jnp.ones((_E,), jnp.float32)
    deg = (G["rowptr"][1:_N + 1] - G["rowptr"][:_N]).astype(jnp.float32)
    valid = jnp.ones((_N,), bool)

    h = _gcn_conv(x, G, ew, deg, W1, b1)
    h = jax.nn.leaky_relu(h, 0.01)
    h, ew, valid, deg = _pool(h, G, ew, valid, 6000, p1_Wr, p1_br, p1_Wroot)

    h = _gcn_conv(h, G, ew, deg, W2, b2)
    h = jax.nn.leaky_relu(h, 0.01)
    h, ew, valid, deg = _pool(h, G, ew, valid, 3600, p2_Wr, p2_br, p2_Wroot)

    h = _gcn_conv(h, G, ew, deg, W3, b3)
    h = jax.nn.leaky_relu(h, 0.01)
    h, ew, valid, deg = _pool(h, G, ew, valid, 1800, p3_Wr, p3_br, p3_Wroot)

    x2 = _gcn_conv(h, G, ew, deg, W4, b4)

    vm = valid[:, None]
    gmax = jnp.max(jnp.where(vm, x2, -jnp.inf), axis=0, keepdims=True)
    gmean = jnp.sum(jnp.where(vm, x2, 0.0), axis=0, keepdims=True) / 1800.0
    return jnp.concatenate([gmax, gmean], axis=1)
